# speedup vs baseline: 21.4788x; 21.4788x over previous
import jax
import jax.numpy as jnp
from jax import lax
from jax.experimental import pallas as pl
from jax.experimental.pallas import tpu as pltpu
from jax.experimental.pallas import tpu_sc as plsc

N = 1048576
NQ = 8192
NB = 2 * NQ
LAM = 0.5
GAMMA = 0.2

NC = 2
NS = 16
NW = NC * NS
CHUNK = N // NW
SUB = 8192
NSUB = CHUNK // SUB
L = 16
SLICE = NB // NS

ROWS = 1024
COLS = 1024
BROWS = 128
GRID = ROWS // BROWS


def _sc_body(scores_hbm, labels_hbm, qids_hbm, out_hbm,
             sc_v, lb_v, qd_v, idx_v, ones_v, sums_sh, cnts_sh):
    cid = lax.axis_index("c")
    sid = lax.axis_index("s")
    wid = cid * NS + sid
    base = wid * CHUNK

    def _fill(i, _):
        ones_v[pl.ds(i * L, L)] = jnp.ones((L,), jnp.float32)
        sc_v[pl.ds(i * L, L)] = jnp.zeros((L,), jnp.float32)
        return 0
    lax.fori_loop(0, SUB // L, _fill, 0)

    pltpu.sync_copy(sc_v.at[pl.ds(0, SLICE)], sums_sh.at[pl.ds(sid * SLICE, SLICE)])
    pltpu.sync_copy(sc_v.at[pl.ds(0, SLICE)], cnts_sh.at[pl.ds(sid * SLICE, SLICE)])
    plsc.subcore_barrier()

    for s in range(NSUB):
        off = base + s * SUB
        pltpu.sync_copy(scores_hbm.at[pl.ds(off, SUB)], sc_v)
        pltpu.sync_copy(labels_hbm.at[pl.ds(off, SUB)], lb_v)
        pltpu.sync_copy(qids_hbm.at[pl.ds(off, SUB)], qd_v)

        def _mkidx(i, _):
            q = qd_v[pl.ds(i * L, L)]
            lb = lb_v[pl.ds(i * L, L)]
            idx_v[pl.ds(i * L, L)] = q + (lb << 13)
            return 0
        lax.fori_loop(0, SUB // L, _mkidx, 0)

        pltpu.sync_copy(sc_v, sums_sh.at[idx_v], add=True)
        pltpu.sync_copy(ones_v, cnts_sh.at[idx_v], add=True)

    plsc.subcore_barrier()
    pltpu.sync_copy(sums_sh.at[pl.ds(sid * SLICE, SLICE)],
                    out_hbm.at[cid, 0, pl.ds(sid * SLICE, SLICE)])
    pltpu.sync_copy(cnts_sh.at[pl.ds(sid * SLICE, SLICE)],
                    out_hbm.at[cid, 1, pl.ds(sid * SLICE, SLICE)])


_sc_call = pl.kernel(
    _sc_body,
    out_type=jax.ShapeDtypeStruct((NC, 2, NB), jnp.float32),
    mesh=plsc.VectorSubcoreMesh(core_axis_name="c", subcore_axis_name="s"),
    scratch_types=[
        pltpu.VMEM((SUB,), jnp.float32),
        pltpu.VMEM((SUB,), jnp.int32),
        pltpu.VMEM((SUB,), jnp.int32),
        pltpu.VMEM((SUB,), jnp.int32),
        pltpu.VMEM((SUB,), jnp.float32),
        pltpu.VMEM_SHARED((NB,), jnp.float32),
        pltpu.VMEM_SHARED((NB,), jnp.float32),
    ],
)


def _tc_body(scores_ref, labels_ref, acc_ref, out_ref, bce_acc):
    i = pl.program_id(0)

    @pl.when(i == 0)
    def _():
        bce_acc[0, 0] = 0.0

    s = scores_ref[...]
    lb = labels_ref[...]
    t = lb * jnp.maximum(jnp.log(s), -100.0) \
        + (1.0 - lb) * jnp.maximum(jnp.log1p(-s), -100.0)
    bce_acc[0, 0] += jnp.sum(t)

    @pl.when(i == GRID - 1)
    def _():
        a = acc_ref[...]
        sums = a[0:1, :] + a[2:3, :]
        cnts = a[1:2, :] + a[3:4, :]
        neg_s = sums[:, :NQ]
        pos_s = sums[:, NQ:]
        neg_c = cnts[:, :NQ]
        pos_c = cnts[:, NQ:]
        valid = (pos_c > 0.0) & (neg_c > 0.0)
        pos_m = pos_s / jnp.maximum(pos_c, 1.0)
        neg_m = neg_s / jnp.maximum(neg_c, 1.0)
        delta = pos_m - neg_m
        terms = jnp.where(valid, jnp.maximum(GAMMA - delta, 0.0), 0.0)
        ng = jnp.sum(valid.astype(jnp.float32))
        ldc = jnp.where(ng > 0.0, jnp.sum(terms) / jnp.maximum(ng, 1.0), 0.0)
        lce = -bce_acc[0, 0] / N
        out_ref[0, 0] = lce + LAM * ldc


def _tc_call(scores2, labels2, acc4):
    return pl.pallas_call(
        _tc_body,
        grid=(GRID,),
        in_specs=[
            pl.BlockSpec((BROWS, COLS), lambda i: (i, 0)),
            pl.BlockSpec((BROWS, COLS), lambda i: (i, 0)),
            pl.BlockSpec((4, NB), lambda i: (0, 0)),
        ],
        out_specs=pl.BlockSpec(memory_space=pltpu.SMEM),
        out_shape=jax.ShapeDtypeStruct((1, 1), jnp.float32),
        scratch_shapes=[pltpu.SMEM((1, 1), jnp.float32)],
    )(scores2, labels2, acc4)


def kernel(scores, labels, qids):
    labels_i = labels.astype(jnp.int32)
    qids_i = qids.astype(jnp.int32)
    acc = _sc_call(scores, labels_i, qids_i)
    acc4 = acc.reshape(4, NB)
    scores2 = scores.reshape(ROWS, COLS)
    labels2 = labels.astype(jnp.float32).reshape(ROWS, COLS)
    out = _tc_call(scores2, labels2, acc4)
    return out[0, 0]


# speedup vs baseline: 22.0117x; 1.0248x over previous
import jax
import jax.numpy as jnp
from jax import lax
from jax.experimental import pallas as pl
from jax.experimental.pallas import tpu as pltpu
from jax.experimental.pallas import tpu_sc as plsc

N = 1048576
NQ = 8192
NB = 2 * NQ
LAM = 0.5
GAMMA = 0.2

NC = 2
NS = 16
NW = NC * NS
CHUNK = N // NW
SUB = 8192
NSUB = CHUNK // SUB
L = 16
SLICE = NB // NS

ROWS = 1024
COLS = 1024
BROWS = 128
GRID = ROWS // BROWS

REP = 8
ACC = REP * NB


def _sc_body(scores_hbm, labels_hbm, qids_hbm, out_hbm,
             sc_v, lb_v, qd_v, idx_v, ones_v, tmp_v, sums_sh, cnts_sh):
    cid = lax.axis_index("c")
    sid = lax.axis_index("s")
    wid = cid * NS + sid
    base = wid * CHUNK

    def _fill(i, _):
        ones_v[pl.ds(i * L, L)] = jnp.ones((L,), jnp.float32)
        sc_v[pl.ds(i * L, L)] = jnp.zeros((L,), jnp.float32)
        return 0
    lax.fori_loop(0, SUB // L, _fill, 0)

    for r in range(REP):
        pltpu.sync_copy(sc_v.at[pl.ds(0, SLICE)],
                        sums_sh.at[pl.ds(r * NB + sid * SLICE, SLICE)])
        pltpu.sync_copy(sc_v.at[pl.ds(0, SLICE)],
                        cnts_sh.at[pl.ds(r * NB + sid * SLICE, SLICE)])
    plsc.subcore_barrier()

    roff = (lax.iota(jnp.int32, L) & (REP - 1)) * NB

    for s in range(NSUB):
        off = base + s * SUB
        pltpu.sync_copy(scores_hbm.at[pl.ds(off, SUB)], sc_v)
        pltpu.sync_copy(labels_hbm.at[pl.ds(off, SUB)], lb_v)
        pltpu.sync_copy(qids_hbm.at[pl.ds(off, SUB)], qd_v)

        def _mkidx(i, _):
            q = qd_v[pl.ds(i * L, L)]
            lb = lb_v[pl.ds(i * L, L)]
            idx_v[pl.ds(i * L, L)] = q + (lb << 13) + roff
            return 0
        lax.fori_loop(0, SUB // L, _mkidx, 0)

        pltpu.sync_copy(sc_v, sums_sh.at[idx_v], add=True)
        pltpu.sync_copy(ones_v, cnts_sh.at[idx_v], add=True)

    plsc.subcore_barrier()

    for (acc_sh, row) in ((sums_sh, 0), (cnts_sh, 1)):
        pltpu.sync_copy(acc_sh.at[pl.ds(sid * SLICE, SLICE)], tmp_v.at[pl.ds(0, SLICE)])
        for r in range(1, REP):
            pltpu.sync_copy(acc_sh.at[pl.ds(r * NB + sid * SLICE, SLICE)],
                            tmp_v.at[pl.ds(r * SLICE, SLICE)])

        def _fold(i, _):
            acc = tmp_v[pl.ds(i * L, L)]
            for r in range(1, REP):
                acc = acc + tmp_v[pl.ds(r * SLICE + i * L, L)]
            tmp_v[pl.ds(i * L, L)] = acc
            return 0
        lax.fori_loop(0, SLICE // L, _fold, 0)
        pltpu.sync_copy(tmp_v.at[pl.ds(0, SLICE)],
                        out_hbm.at[cid, row, pl.ds(sid * SLICE, SLICE)])


_sc_call = pl.kernel(
    _sc_body,
    out_type=jax.ShapeDtypeStruct((NC, 2, NB), jnp.float32),
    mesh=plsc.VectorSubcoreMesh(core_axis_name="c", subcore_axis_name="s"),
    scratch_types=[
        pltpu.VMEM((SUB,), jnp.float32),
        pltpu.VMEM((SUB,), jnp.int32),
        pltpu.VMEM((SUB,), jnp.int32),
        pltpu.VMEM((SUB,), jnp.int32),
        pltpu.VMEM((SUB,), jnp.float32),
        pltpu.VMEM((REP * SLICE,), jnp.float32),
        pltpu.VMEM_SHARED((ACC,), jnp.float32),
        pltpu.VMEM_SHARED((ACC,), jnp.float32),
    ],
)


def _tc_body(scores_ref, labels_ref, acc_ref, out_ref, bce_acc):
    i = pl.program_id(0)

    @pl.when(i == 0)
    def _():
        bce_acc[0, 0] = 0.0

    s = scores_ref[...]
    lb = labels_ref[...]
    t = lb * jnp.maximum(jnp.log(s), -100.0) \
        + (1.0 - lb) * jnp.maximum(jnp.log1p(-s), -100.0)
    bce_acc[0, 0] += jnp.sum(t)

    @pl.when(i == GRID - 1)
    def _():
        a = acc_ref[...]
        sums = a[0:1, :] + a[2:3, :]
        cnts = a[1:2, :] + a[3:4, :]
        neg_s = sums[:, :NQ]
        pos_s = sums[:, NQ:]
        neg_c = cnts[:, :NQ]
        pos_c = cnts[:, NQ:]
        valid = (pos_c > 0.0) & (neg_c > 0.0)
        pos_m = pos_s / jnp.maximum(pos_c, 1.0)
        neg_m = neg_s / jnp.maximum(neg_c, 1.0)
        delta = pos_m - neg_m
        terms = jnp.where(valid, jnp.maximum(GAMMA - delta, 0.0), 0.0)
        ng = jnp.sum(valid.astype(jnp.float32))
        ldc = jnp.where(ng > 0.0, jnp.sum(terms) / jnp.maximum(ng, 1.0), 0.0)
        lce = -bce_acc[0, 0] / N
        out_ref[0, 0] = lce + LAM * ldc


def _tc_call(scores2, labels2, acc4):
    return pl.pallas_call(
        _tc_body,
        grid=(GRID,),
        in_specs=[
            pl.BlockSpec((BROWS, COLS), lambda i: (i, 0)),
            pl.BlockSpec((BROWS, COLS), lambda i: (i, 0)),
            pl.BlockSpec((4, NB), lambda i: (0, 0)),
        ],
        out_specs=pl.BlockSpec(memory_space=pltpu.SMEM),
        out_shape=jax.ShapeDtypeStruct((1, 1), jnp.float32),
        scratch_shapes=[pltpu.SMEM((1, 1), jnp.float32)],
    )(scores2, labels2, acc4)


def kernel(scores, labels, qids):
    labels_i = labels.astype(jnp.int32)
    qids_i = qids.astype(jnp.int32)
    acc = _sc_call(scores, labels_i, qids_i)
    acc4 = acc.reshape(4, NB)
    scores2 = scores.reshape(ROWS, COLS)
    labels2 = labels.astype(jnp.float32).reshape(ROWS, COLS)
    out = _tc_call(scores2, labels2, acc4)
    return out[0, 0]


# speedup vs baseline: 26.0224x; 1.1822x over previous
import jax
import jax.numpy as jnp
from jax import lax
from jax.experimental import pallas as pl
from jax.experimental.pallas import tpu as pltpu
from jax.experimental.pallas import tpu_sc as plsc

N = 1048576
NQ = 8192
NB = 2 * NQ
LAM = 0.5
GAMMA = 0.2

NC = 2
NS = 16
NW = NC * NS
CHUNK = N // NW
SUB = 8192
NSUB = CHUNK // SUB
L = 16
SLICE = NB // NS

ROWS = 1024
COLS = 1024
BROWS = 128
GRID = ROWS // BROWS

REP = 8
ACC = REP * NB


def _sc_body(scores_hbm, labels_hbm, qids_hbm, out_hbm,
             sc_v0, lb_v0, qd_v0, idx_v0, sc_v1, lb_v1, qd_v1, idx_v1,
             ones_v, tmp_v, sums_sh, cnts_sh,
             in_sem0, in_sem1, st_sem0, st_sem1):
    cid = lax.axis_index("c")
    sid = lax.axis_index("s")
    wid = cid * NS + sid
    base = wid * CHUNK

    sc_v = (sc_v0, sc_v1)
    lb_v = (lb_v0, lb_v1)
    qd_v = (qd_v0, qd_v1)
    idx_v = (idx_v0, idx_v1)
    in_sem = (in_sem0, in_sem1)
    st_sem = (st_sem0, st_sem1)

    @plsc.parallel_loop(0, SLICE // L, 1, unroll=8)
    def _zfill(i):
        tmp_v[pl.ds(i * L, L)] = jnp.zeros((L,), jnp.float32)

    for r in range(REP):
        pltpu.sync_copy(tmp_v.at[pl.ds(0, SLICE)],
                        sums_sh.at[pl.ds(r * NB + sid * SLICE, SLICE)])
        pltpu.sync_copy(tmp_v.at[pl.ds(0, SLICE)],
                        cnts_sh.at[pl.ds(r * NB + sid * SLICE, SLICE)])

    def _issue_in(s, b):
        off = base + s * SUB
        return (
            pltpu.async_copy(scores_hbm.at[pl.ds(off, SUB)], sc_v[b], in_sem[b]),
            pltpu.async_copy(labels_hbm.at[pl.ds(off, SUB)], lb_v[b], in_sem[b]),
            pltpu.async_copy(qids_hbm.at[pl.ds(off, SUB)], qd_v[b], in_sem[b]),
        )

    pending_in = _issue_in(0, 0)

    @plsc.parallel_loop(0, SUB // L, 1, unroll=8)
    def _ofill(i):
        ones_v[pl.ds(i * L, L)] = jnp.ones((L,), jnp.float32)

    plsc.subcore_barrier()

    roff = (lax.iota(jnp.int32, L) & (REP - 1)) * NB

    pending_st = [None, None]
    for s in range(NSUB):
        b = s & 1
        for d in pending_in:
            d.wait()
        if s + 1 < NSUB:
            if pending_st[b ^ 1] is not None:
                for d in pending_st[b ^ 1]:
                    d.wait()
                pending_st[b ^ 1] = None
            pending_in = _issue_in(s + 1, b ^ 1)

        qd_b, lb_b, idx_b, sc_b = qd_v[b], lb_v[b], idx_v[b], sc_v[b]

        @plsc.parallel_loop(0, SUB // L, 1, unroll=8)
        def _mkidx(i):
            q = qd_b[pl.ds(i * L, L)]
            lb = lb_b[pl.ds(i * L, L)]
            idx_b[pl.ds(i * L, L)] = q + (lb << 13) + roff

        pending_st[b] = (
            pltpu.async_copy(sc_b, sums_sh.at[idx_b], st_sem[b], add=True),
            pltpu.async_copy(ones_v, cnts_sh.at[idx_b], st_sem[b], add=True),
        )

    for b in (0, 1):
        if pending_st[b] is not None:
            for d in pending_st[b]:
                d.wait()

    plsc.subcore_barrier()

    for (acc_sh, row) in ((sums_sh, 0), (cnts_sh, 1)):
        pltpu.sync_copy(acc_sh.at[pl.ds(sid * SLICE, SLICE)], tmp_v.at[pl.ds(0, SLICE)])
        for r in range(1, REP):
            pltpu.sync_copy(acc_sh.at[pl.ds(r * NB + sid * SLICE, SLICE)],
                            tmp_v.at[pl.ds(r * SLICE, SLICE)])

        def _fold(i, _):
            acc = tmp_v[pl.ds(i * L, L)]
            for r in range(1, REP):
                acc = acc + tmp_v[pl.ds(r * SLICE + i * L, L)]
            tmp_v[pl.ds(i * L, L)] = acc
            return 0
        lax.fori_loop(0, SLICE // L, _fold, 0)
        pltpu.sync_copy(tmp_v.at[pl.ds(0, SLICE)],
                        out_hbm.at[cid, row, pl.ds(sid * SLICE, SLICE)])


_sc_call = pl.kernel(
    _sc_body,
    out_type=jax.ShapeDtypeStruct((NC, 2, NB), jnp.float32),
    mesh=plsc.VectorSubcoreMesh(core_axis_name="c", subcore_axis_name="s"),
    scratch_types=[
        pltpu.VMEM((SUB,), jnp.float32),
        pltpu.VMEM((SUB,), jnp.int32),
        pltpu.VMEM((SUB,), jnp.int32),
        pltpu.VMEM((SUB,), jnp.int32),
        pltpu.VMEM((SUB,), jnp.float32),
        pltpu.VMEM((SUB,), jnp.int32),
        pltpu.VMEM((SUB,), jnp.int32),
        pltpu.VMEM((SUB,), jnp.int32),
        pltpu.VMEM((SUB,), jnp.float32),
        pltpu.VMEM((REP * SLICE,), jnp.float32),
        pltpu.VMEM_SHARED((ACC,), jnp.float32),
        pltpu.VMEM_SHARED((ACC,), jnp.float32),
        pltpu.SemaphoreType.DMA,
        pltpu.SemaphoreType.DMA,
        pltpu.SemaphoreType.DMA,
        pltpu.SemaphoreType.DMA,
    ],
)


def _tc_body(scores_ref, labels_ref, acc_ref, out_ref, bce_acc):
    i = pl.program_id(0)

    @pl.when(i == 0)
    def _():
        bce_acc[0, 0] = 0.0

    s = scores_ref[...]
    lb = labels_ref[...]
    t = lb * jnp.maximum(jnp.log(s), -100.0) \
        + (1.0 - lb) * jnp.maximum(jnp.log1p(-s), -100.0)
    bce_acc[0, 0] += jnp.sum(t)

    @pl.when(i == GRID - 1)
    def _():
        a = acc_ref[...]
        sums = a[0:1, :] + a[2:3, :]
        cnts = a[1:2, :] + a[3:4, :]
        neg_s = sums[:, :NQ]
        pos_s = sums[:, NQ:]
        neg_c = cnts[:, :NQ]
        pos_c = cnts[:, NQ:]
        valid = (pos_c > 0.0) & (neg_c > 0.0)
        pos_m = pos_s / jnp.maximum(pos_c, 1.0)
        neg_m = neg_s / jnp.maximum(neg_c, 1.0)
        delta = pos_m - neg_m
        terms = jnp.where(valid, jnp.maximum(GAMMA - delta, 0.0), 0.0)
        ng = jnp.sum(valid.astype(jnp.float32))
        ldc = jnp.where(ng > 0.0, jnp.sum(terms) / jnp.maximum(ng, 1.0), 0.0)
        lce = -bce_acc[0, 0] / N
        out_ref[0, 0] = lce + LAM * ldc


def _tc_call(scores2, labels2, acc4):
    return pl.pallas_call(
        _tc_body,
        grid=(GRID,),
        in_specs=[
            pl.BlockSpec((BROWS, COLS), lambda i: (i, 0)),
            pl.BlockSpec((BROWS, COLS), lambda i: (i, 0)),
            pl.BlockSpec((4, NB), lambda i: (0, 0)),
        ],
        out_specs=pl.BlockSpec(memory_space=pltpu.SMEM),
        out_shape=jax.ShapeDtypeStruct((1, 1), jnp.float32),
        scratch_shapes=[pltpu.SMEM((1, 1), jnp.float32)],
    )(scores2, labels2, acc4)


def kernel(scores, labels, qids):
    labels_i = labels.astype(jnp.int32)
    qids_i = qids.astype(jnp.int32)
    acc = _sc_call(scores, labels_i, qids_i)
    acc4 = acc.reshape(4, NB)
    scores2 = scores.reshape(ROWS, COLS)
    labels2 = labels.astype(jnp.float32).reshape(ROWS, COLS)
    out = _tc_call(scores2, labels2, acc4)
    return out[0, 0]


# speedup vs baseline: 30.6461x; 1.1777x over previous
import jax
import jax.numpy as jnp
from jax import lax
from jax.experimental import pallas as pl
from jax.experimental.pallas import tpu as pltpu
from jax.experimental.pallas import tpu_sc as plsc

N = 1048576
NQ = 8192
NB = 2 * NQ
LAM = 0.5
GAMMA = 0.2

NC = 2
NS = 16
NW = NC * NS
CHUNK = N // NW
SUB = 8192
NSUB = CHUNK // SUB
L = 16
SLICE = NB // NS

ROWS = 1024
COLS = 1024
BROWS = 128
GRID = ROWS // BROWS

REP = 8
ACC = REP * NB


def _sc_body(scores_hbm, labels_hbm, qids_hbm, out_hbm,
             sc_v0, lb_v0, qd_v0, idx_v0, sc_v1, lb_v1, qd_v1, idx_v1,
             ones_v, tmp_v, sums_sh, cnts_sh,
             in_sem0, in_sem1, st_sem0, st_sem1):
    cid = lax.axis_index("c")
    sid = lax.axis_index("s")
    wid = cid * NS + sid
    base = wid * CHUNK

    sc_v = (sc_v0, sc_v1)
    lb_v = (lb_v0, lb_v1)
    qd_v = (qd_v0, qd_v1)
    idx_v = (idx_v0, idx_v1)
    in_sem = (in_sem0, in_sem1)
    st_sem = (st_sem0, st_sem1)

    @plsc.parallel_loop(0, SLICE // L, 1, unroll=8)
    def _zfill(i):
        tmp_v[pl.ds(i * L, L)] = jnp.zeros((L,), jnp.float32)

    for r in range(REP):
        pltpu.sync_copy(tmp_v.at[pl.ds(0, SLICE)],
                        sums_sh.at[pl.ds(r * NB + sid * SLICE, SLICE)])
        pltpu.sync_copy(tmp_v.at[pl.ds(0, SLICE)],
                        cnts_sh.at[pl.ds(r * NB + sid * SLICE, SLICE)])

    def _issue_in(s, b):
        off = base + s * SUB
        return (
            pltpu.async_copy(scores_hbm.at[pl.ds(off, SUB)], sc_v[b], in_sem[b]),
            pltpu.async_copy(labels_hbm.at[pl.ds(off, SUB)], lb_v[b], in_sem[b]),
            pltpu.async_copy(qids_hbm.at[pl.ds(off, SUB)], qd_v[b], in_sem[b]),
        )

    pending_in = _issue_in(0, 0)

    @plsc.parallel_loop(0, SUB // L, 1, unroll=8)
    def _ofill(i):
        ones_v[pl.ds(i * L, L)] = jnp.ones((L,), jnp.float32)

    plsc.subcore_barrier()

    roff = (lax.iota(jnp.int32, L) & (REP - 1)) * NB

    pending_st = [None, None]
    for s in range(NSUB):
        b = s & 1
        for d in pending_in:
            d.wait()
        if s + 1 < NSUB:
            if pending_st[b ^ 1] is not None:
                for d in pending_st[b ^ 1]:
                    d.wait()
                pending_st[b ^ 1] = None
            pending_in = _issue_in(s + 1, b ^ 1)

        qd_b, lb_b, idx_b, sc_b = qd_v[b], lb_v[b], idx_v[b], sc_v[b]

        @plsc.parallel_loop(0, SUB // L, 1, unroll=8)
        def _mkidx(i):
            q = qd_b[pl.ds(i * L, L)]
            lb = lb_b[pl.ds(i * L, L)]
            idx_b[pl.ds(i * L, L)] = q + (lb << 13) + roff

        pending_st[b] = (
            pltpu.async_copy(sc_b, sums_sh.at[idx_b], st_sem[b], add=True),
            pltpu.async_copy(ones_v, cnts_sh.at[idx_b], st_sem[b], add=True),
        )

    for b in (0, 1):
        if pending_st[b] is not None:
            for d in pending_st[b]:
                d.wait()

    plsc.subcore_barrier()

    for (acc_sh, row) in ((sums_sh, 0), (cnts_sh, 1)):
        pltpu.sync_copy(acc_sh.at[pl.ds(sid * SLICE, SLICE)], tmp_v.at[pl.ds(0, SLICE)])
        for r in range(1, REP):
            pltpu.sync_copy(acc_sh.at[pl.ds(r * NB + sid * SLICE, SLICE)],
                            tmp_v.at[pl.ds(r * SLICE, SLICE)])

        def _fold(i, _):
            acc = tmp_v[pl.ds(i * L, L)]
            for r in range(1, REP):
                acc = acc + tmp_v[pl.ds(r * SLICE + i * L, L)]
            tmp_v[pl.ds(i * L, L)] = acc
            return 0
        lax.fori_loop(0, SLICE // L, _fold, 0)
        pltpu.sync_copy(tmp_v.at[pl.ds(0, SLICE)],
                        out_hbm.at[cid, row, pl.ds(sid * SLICE, SLICE)])


_sc_call = pl.kernel(
    _sc_body,
    out_type=jax.ShapeDtypeStruct((NC, 2, NB), jnp.float32),
    mesh=plsc.VectorSubcoreMesh(core_axis_name="c", subcore_axis_name="s"),
    scratch_types=[
        pltpu.VMEM((SUB,), jnp.float32),
        pltpu.VMEM((SUB,), jnp.int32),
        pltpu.VMEM((SUB,), jnp.int32),
        pltpu.VMEM((SUB,), jnp.int32),
        pltpu.VMEM((SUB,), jnp.float32),
        pltpu.VMEM((SUB,), jnp.int32),
        pltpu.VMEM((SUB,), jnp.int32),
        pltpu.VMEM((SUB,), jnp.int32),
        pltpu.VMEM((SUB,), jnp.float32),
        pltpu.VMEM((REP * SLICE,), jnp.float32),
        pltpu.VMEM_SHARED((ACC,), jnp.float32),
        pltpu.VMEM_SHARED((ACC,), jnp.float32),
        pltpu.SemaphoreType.DMA,
        pltpu.SemaphoreType.DMA,
        pltpu.SemaphoreType.DMA,
        pltpu.SemaphoreType.DMA,
    ],
)


def _bce_body(scores_ref, labels_ref, out_ref):
    i = pl.program_id(0)

    @pl.when(i == 0)
    def _():
        out_ref[0, 0] = 0.0

    s = scores_ref[...]
    lb = labels_ref[...]
    t = lb * jnp.maximum(jnp.log(s), -100.0) \
        + (1.0 - lb) * jnp.maximum(jnp.log1p(-s), -100.0)
    out_ref[0, 0] += jnp.sum(t)


def _bce_call(scores2, labels2):
    return pl.pallas_call(
        _bce_body,
        grid=(GRID,),
        in_specs=[
            pl.BlockSpec((BROWS, COLS), lambda i: (i, 0)),
            pl.BlockSpec((BROWS, COLS), lambda i: (i, 0)),
        ],
        out_specs=pl.BlockSpec(memory_space=pltpu.SMEM),
        out_shape=jax.ShapeDtypeStruct((1, 1), jnp.float32),
    )(scores2, labels2)


def _comb_body(acc_ref, bce_ref, out_ref):
    a = acc_ref[...]
    sums = a[0:1, :] + a[2:3, :]
    cnts = a[1:2, :] + a[3:4, :]
    neg_s = sums[:, :NQ]
    pos_s = sums[:, NQ:]
    neg_c = cnts[:, :NQ]
    pos_c = cnts[:, NQ:]
    valid = (pos_c > 0.0) & (neg_c > 0.0)
    pos_m = pos_s / jnp.maximum(pos_c, 1.0)
    neg_m = neg_s / jnp.maximum(neg_c, 1.0)
    delta = pos_m - neg_m
    terms = jnp.where(valid, jnp.maximum(GAMMA - delta, 0.0), 0.0)
    ng = jnp.sum(valid.astype(jnp.float32))
    ldc = jnp.where(ng > 0.0, jnp.sum(terms) / jnp.maximum(ng, 1.0), 0.0)
    lce = -bce_ref[0, 0] / N
    out_ref[0, 0] = lce + LAM * ldc


def _comb_call(acc4, bce):
    return pl.pallas_call(
        _comb_body,
        in_specs=[
            pl.BlockSpec((4, NB), lambda: (0, 0)),
            pl.BlockSpec(memory_space=pltpu.SMEM),
        ],
        out_specs=pl.BlockSpec(memory_space=pltpu.SMEM),
        out_shape=jax.ShapeDtypeStruct((1, 1), jnp.float32),
    )(acc4, bce)


def kernel(scores, labels, qids):
    labels_i = labels.astype(jnp.int32)
    qids_i = qids.astype(jnp.int32)
    acc = _sc_call(scores, labels_i, qids_i)
    acc4 = acc.reshape(4, NB)
    scores2 = scores.reshape(ROWS, COLS)
    labels2 = labels.astype(jnp.float32).reshape(ROWS, COLS)
    bce = _bce_call(scores2, labels2)
    out = _comb_call(acc4, bce)
    return out[0, 0]
